# baseline (speedup 1.0000x reference)
import functools

import jax
import jax.numpy as jnp
import numpy as np
from jax import lax
from jax.experimental import pallas as pl
from jax.experimental.pallas import tpu as pltpu
from jax.experimental.pallas import tpu_sc as plsc

_MASK_RATE = 0.5
_REPLACE_RATE = 0.05
_MASK_TOKEN_RATE = 1.0 - _REPLACE_RATE

_NC = 2
_NS = 16
_NW = _NC * _NS
_R = 800


@functools.lru_cache(maxsize=None)
def _plan(num_nodes: int):
    num_mask = int(_MASK_RATE * num_nodes)
    cpu = jax.local_devices(backend="cpu")[0]
    with jax.ensure_compile_time_eval(), jax.default_device(cpu):
        key = jax.random.key(42)
        kp, km, kn = jax.random.split(key, 3)
        perm = np.asarray(jax.random.permutation(kp, num_nodes))
        perm_mask = np.asarray(jax.random.permutation(km, num_mask))
        noise_all = np.asarray(jax.random.permutation(kn, num_nodes))
    mask_nodes = perm[:num_mask]
    keep_nodes = perm[num_mask:]
    num_noise = int(_REPLACE_RATE * num_mask)
    num_token = int(_MASK_TOKEN_RATE * num_mask)
    token_nodes = mask_nodes[perm_mask[:num_token]]
    noise_nodes = mask_nodes[perm_mask[num_mask - num_noise:]]
    noise_src = noise_all[:num_noise]

    assert np.intersect1d(token_nodes, noise_nodes).size == 0
    assert num_nodes % _R == 0

    is_token = np.zeros(num_nodes, dtype=bool)
    is_token[token_nodes] = True
    mask8 = np.broadcast_to(is_token[:, None], (num_nodes, 128)).astype(np.int8)

    order = np.argsort(noise_nodes)
    ndst = noise_nodes[order].astype(np.int32)
    nsrc = noise_src[order].astype(np.int32)
    nblocks = num_nodes // _R
    lo = np.searchsorted(ndst, np.arange(nblocks + 1) * _R).astype(np.int32)
    dst_local = (ndst % _R).astype(np.int32)

    per = 8 * _NW
    nvp = -(-num_noise // per) * per
    nsrc_pad = np.full(nvp, nsrc[0], dtype=np.int32)
    nsrc_pad[:num_noise] = nsrc
    dst_local_pad = np.zeros(nvp, dtype=np.int32)
    dst_local_pad[:num_noise] = dst_local

    return dict(
        mask_nodes=mask_nodes.astype(np.int32),
        keep_nodes=keep_nodes.astype(np.int32),
        mask8=mask8, nsrc=nsrc_pad, lo=lo, dst_local=dst_local_pad, nvp=nvp,
    )


def _sc_gather_rows(x, sidx, nvp):
    d = x.shape[1]
    bpw = nvp // _NW
    mesh = plsc.VectorSubcoreMesh(core_axis_name="c", subcore_axis_name="s")

    @functools.partial(
        pl.kernel,
        out_type=jax.ShapeDtypeStruct((nvp, d), x.dtype),
        mesh=mesh,
        scratch_types=[
            pltpu.VMEM((bpw,), jnp.int32),
            pltpu.VMEM((bpw, d), jnp.float32),
            pltpu.SemaphoreType.DMA,
        ],
    )
    def g(x_hbm, sidx_hbm, out_hbm, idx_v, rows_v, sem):
        wid = lax.axis_index("s") * _NC + lax.axis_index("c")
        base = wid * bpw
        pltpu.sync_copy(sidx_hbm.at[pl.ds(base, bpw)], idx_v)
        pltpu.async_copy(x_hbm.at[idx_v], rows_v, sem).wait()
        pltpu.sync_copy(rows_v, out_hbm.at[pl.ds(base, bpw)])

    return g(x, sidx)


def _tc_select_patch(x, mask8, tok, noise_vals, lo, dst_local):
    num_nodes, d = x.shape
    nvp = noise_vals.shape[0]
    nblocks = num_nodes // _R

    def body(x_ref, m_ref, tok_ref, nv_ref, lo_ref, dl_ref, o_ref):
        b = pl.program_id(0)
        m = m_ref[...].astype(jnp.float32)
        o_ref[...] = x_ref[...] * (1.0 - m) + tok_ref[...] * m

        def patch(j, carry):
            s = dl_ref[j]
            o_ref[pl.ds(s, 1), :] = nv_ref[pl.ds(j, 1), :]
            return carry

        lax.fori_loop(lo_ref[b], lo_ref[b + 1], patch, 0)

    return pl.pallas_call(
        body,
        grid=(nblocks,),
        in_specs=[
            pl.BlockSpec((_R, d), lambda b: (b, 0)),
            pl.BlockSpec((_R, d), lambda b: (b, 0)),
            pl.BlockSpec((1, d), lambda b: (0, 0)),
            pl.BlockSpec((nvp, d), lambda b: (0, 0)),
            pl.BlockSpec(memory_space=pltpu.SMEM),
            pl.BlockSpec(memory_space=pltpu.SMEM),
        ],
        out_specs=pl.BlockSpec((_R, d), lambda b: (b, 0)),
        out_shape=jax.ShapeDtypeStruct((num_nodes, d), x.dtype),
    )(x, mask8, tok, noise_vals, lo, dst_local)


def kernel(adj, x, enc_mask_token):
    p = _plan(x.shape[0])
    noise_vals = _sc_gather_rows(x, jnp.asarray(p["nsrc"]), p["nvp"])
    out_x = _tc_select_patch(
        x, jnp.asarray(p["mask8"]), enc_mask_token, noise_vals,
        jnp.asarray(p["lo"]), jnp.asarray(p["dst_local"]),
    )
    return (adj, out_x, jnp.asarray(p["mask_nodes"]), jnp.asarray(p["keep_nodes"]))


# speedup vs baseline: 1.6938x; 1.6938x over previous
import functools

import jax
import jax.numpy as jnp
import numpy as np
from jax import lax
from jax.experimental import pallas as pl
from jax.experimental.pallas import tpu as pltpu
from jax.experimental.pallas import tpu_sc as plsc

_MASK_RATE = 0.5
_REPLACE_RATE = 0.05
_MASK_TOKEN_RATE = 1.0 - _REPLACE_RATE

_NC = 2
_NS = 16
_NW = _NC * _NS
_R = 20000


@functools.lru_cache(maxsize=None)
def _plan(num_nodes: int):
    num_mask = int(_MASK_RATE * num_nodes)
    cpu = jax.local_devices(backend="cpu")[0]
    with jax.ensure_compile_time_eval(), jax.default_device(cpu):
        key = jax.random.key(42)
        kp, km, kn = jax.random.split(key, 3)
        perm = np.asarray(jax.random.permutation(kp, num_nodes))
        perm_mask = np.asarray(jax.random.permutation(km, num_mask))
        noise_all = np.asarray(jax.random.permutation(kn, num_nodes))
    mask_nodes = perm[:num_mask]
    keep_nodes = perm[num_mask:]
    num_noise = int(_REPLACE_RATE * num_mask)
    num_token = int(_MASK_TOKEN_RATE * num_mask)
    token_nodes = mask_nodes[perm_mask[:num_token]]
    noise_nodes = mask_nodes[perm_mask[num_mask - num_noise:]]
    noise_src = noise_all[:num_noise]

    assert np.intersect1d(token_nodes, noise_nodes).size == 0
    assert num_nodes % _R == 0

    is_token = np.zeros(num_nodes, dtype=bool)
    is_token[token_nodes] = True
    mask8 = np.broadcast_to(is_token[:, None], (num_nodes, 128)).astype(np.int8)

    order = np.argsort(noise_nodes)
    ndst = noise_nodes[order].astype(np.int32)
    nsrc = noise_src[order].astype(np.int32)
    nblocks = num_nodes // _R
    lo = np.searchsorted(ndst, np.arange(nblocks + 1) * _R).astype(np.int32)
    dst_local = (ndst % _R).astype(np.int32)

    per = 8 * _NW
    nvp = -(-num_noise // per) * per
    nsrc_pad = np.full(nvp, nsrc[0], dtype=np.int32)
    nsrc_pad[:num_noise] = nsrc
    dst_local_pad = np.zeros(nvp, dtype=np.int32)
    dst_local_pad[:num_noise] = dst_local

    return dict(
        mask_nodes=mask_nodes.astype(np.int32),
        keep_nodes=keep_nodes.astype(np.int32),
        mask8=mask8, nsrc=nsrc_pad, lo=lo, dst_local=dst_local_pad, nvp=nvp,
    )


def _sc_gather_rows(x, sidx, nvp):
    d = x.shape[1]
    bpw = nvp // _NW
    mesh = plsc.VectorSubcoreMesh(core_axis_name="c", subcore_axis_name="s")

    @functools.partial(
        pl.kernel,
        out_type=jax.ShapeDtypeStruct((nvp, d), x.dtype),
        mesh=mesh,
        scratch_types=[
            pltpu.VMEM((bpw,), jnp.int32),
            pltpu.VMEM((bpw, d), jnp.float32),
            pltpu.SemaphoreType.DMA,
        ],
    )
    def g(x_hbm, sidx_hbm, out_hbm, idx_v, rows_v, sem):
        wid = lax.axis_index("s") * _NC + lax.axis_index("c")
        base = wid * bpw
        pltpu.sync_copy(sidx_hbm.at[pl.ds(base, bpw)], idx_v)
        pltpu.async_copy(x_hbm.at[idx_v], rows_v, sem).wait()
        pltpu.sync_copy(rows_v, out_hbm.at[pl.ds(base, bpw)])

    return g(x, sidx)


def _tc_select_patch(x, mask8, tok, noise_vals, lo, dst_local):
    num_nodes, d = x.shape
    nvp = noise_vals.shape[0]
    nblocks = num_nodes // _R

    def body(x_ref, m_ref, tok_ref, nv_ref, lo_ref, dl_ref, o_ref):
        b = pl.program_id(0)
        m = m_ref[...].astype(jnp.float32)
        o_ref[...] = x_ref[...] * (1.0 - m) + tok_ref[...] * m

        def patch(j, carry):
            s = dl_ref[j]
            o_ref[pl.ds(s, 1), :] = nv_ref[pl.ds(j, 1), :]
            return carry

        lax.fori_loop(lo_ref[b], lo_ref[b + 1], patch, 0)

    return pl.pallas_call(
        body,
        grid=(nblocks,),
        in_specs=[
            pl.BlockSpec((_R, d), lambda b: (b, 0)),
            pl.BlockSpec((_R, d), lambda b: (b, 0)),
            pl.BlockSpec((1, d), lambda b: (0, 0)),
            pl.BlockSpec((nvp, d), lambda b: (0, 0)),
            pl.BlockSpec(memory_space=pltpu.SMEM),
            pl.BlockSpec(memory_space=pltpu.SMEM),
        ],
        out_specs=pl.BlockSpec((_R, d), lambda b: (b, 0)),
        out_shape=jax.ShapeDtypeStruct((num_nodes, d), x.dtype),
    )(x, mask8, tok, noise_vals, lo, dst_local)


def kernel(adj, x, enc_mask_token):
    p = _plan(x.shape[0])
    noise_vals = _sc_gather_rows(x, jnp.asarray(p["nsrc"]), p["nvp"])
    out_x = _tc_select_patch(
        x, jnp.asarray(p["mask8"]), enc_mask_token, noise_vals,
        jnp.asarray(p["lo"]), jnp.asarray(p["dst_local"]),
    )
    return (adj, out_x, jnp.asarray(p["mask_nodes"]), jnp.asarray(p["keep_nodes"]))


# speedup vs baseline: 1.7641x; 1.0415x over previous
import functools

import jax
import jax.numpy as jnp
import numpy as np
from jax import lax
from jax.experimental import pallas as pl
from jax.experimental.pallas import tpu as pltpu
from jax.experimental.pallas import tpu_sc as plsc

_MASK_RATE = 0.5
_REPLACE_RATE = 0.05
_MASK_TOKEN_RATE = 1.0 - _REPLACE_RATE

_NC = 2
_NS = 16
_NW = _NC * _NS
_R = 12800


@functools.lru_cache(maxsize=None)
def _plan(num_nodes: int):
    num_mask = int(_MASK_RATE * num_nodes)
    cpu = jax.local_devices(backend="cpu")[0]
    with jax.ensure_compile_time_eval(), jax.default_device(cpu):
        key = jax.random.key(42)
        kp, km, kn = jax.random.split(key, 3)
        perm = np.asarray(jax.random.permutation(kp, num_nodes))
        perm_mask = np.asarray(jax.random.permutation(km, num_mask))
        noise_all = np.asarray(jax.random.permutation(kn, num_nodes))
    mask_nodes = perm[:num_mask]
    keep_nodes = perm[num_mask:]
    num_noise = int(_REPLACE_RATE * num_mask)
    num_token = int(_MASK_TOKEN_RATE * num_mask)
    token_nodes = mask_nodes[perm_mask[:num_token]]
    noise_nodes = mask_nodes[perm_mask[num_mask - num_noise:]]
    noise_src = noise_all[:num_noise]

    assert np.intersect1d(token_nodes, noise_nodes).size == 0
    assert num_nodes % 32 == 0
    nblocks = -(-num_nodes // _R)

    is_token = np.zeros(num_nodes, dtype=bool)
    is_token[token_nodes] = True
    words = (is_token.reshape(-1, 32).astype(np.uint64)
             << np.arange(32, dtype=np.uint64)).sum(axis=1)
    words = (words & 0xFFFFFFFF).astype(np.uint32).view(np.int32)
    wrows = nblocks * (_R // 32)
    words_pad = np.zeros(wrows, dtype=np.int32)
    words_pad[: words.size] = words
    maskp = np.ascontiguousarray(np.broadcast_to(words_pad[:, None], (wrows, 128)))

    order = np.argsort(noise_nodes)
    ndst = noise_nodes[order].astype(np.int32)
    nsrc = noise_src[order].astype(np.int32)
    lo = np.searchsorted(ndst, np.arange(nblocks + 1) * _R).astype(np.int32)
    dst_local = (ndst % _R).astype(np.int32)

    per = 8 * _NW
    nvp = -(-num_noise // per) * per
    nsrc_pad = np.full(nvp, nsrc[0], dtype=np.int32)
    nsrc_pad[:num_noise] = nsrc
    dst_local_pad = np.zeros(nvp, dtype=np.int32)
    dst_local_pad[:num_noise] = dst_local

    return dict(
        mask_nodes=mask_nodes.astype(np.int32),
        keep_nodes=keep_nodes.astype(np.int32),
        maskp=maskp, nsrc=nsrc_pad, lo=lo, dst_local=dst_local_pad, nvp=nvp,
    )


def _sc_gather_rows(x, sidx, nvp):
    d = x.shape[1]
    bpw = nvp // _NW
    mesh = plsc.VectorSubcoreMesh(core_axis_name="c", subcore_axis_name="s")

    @functools.partial(
        pl.kernel,
        out_type=jax.ShapeDtypeStruct((nvp, d), x.dtype),
        mesh=mesh,
        scratch_types=[
            pltpu.VMEM((bpw,), jnp.int32),
            pltpu.VMEM((bpw, d), jnp.float32),
            pltpu.SemaphoreType.DMA,
        ],
    )
    def g(x_hbm, sidx_hbm, out_hbm, idx_v, rows_v, sem):
        wid = lax.axis_index("s") * _NC + lax.axis_index("c")
        base = wid * bpw
        pltpu.sync_copy(sidx_hbm.at[pl.ds(base, bpw)], idx_v)
        pltpu.async_copy(x_hbm.at[idx_v], rows_v, sem).wait()
        pltpu.sync_copy(rows_v, out_hbm.at[pl.ds(base, bpw)])

    return g(x, sidx)


def _tc_select_patch(x, maskp, tok, noise_vals, lo, dst_local):
    num_nodes, d = x.shape
    nvp = noise_vals.shape[0]
    nblocks = -(-num_nodes // _R)

    def body(x_ref, m_ref, tok_ref, nv_ref, lo_ref, dl_ref, o_ref):
        b = pl.program_id(0)
        words = lax.broadcast_in_dim(m_ref[...], (_R // 32, 32, d), (0, 2))
        bit = lax.broadcasted_iota(jnp.int32, (_R // 32, 32, d), 1)
        m = ((words >> bit) & 1).reshape(_R, d).astype(jnp.float32)
        o_ref[...] = x_ref[...] * (1.0 - m) + tok_ref[...] * m

        def patch(j, carry):
            s = dl_ref[j]
            o_ref[pl.ds(s, 1), :] = nv_ref[pl.ds(j, 1), :]
            return carry

        lax.fori_loop(lo_ref[b], lo_ref[b + 1], patch, 0)

    return pl.pallas_call(
        body,
        grid=(nblocks,),
        in_specs=[
            pl.BlockSpec((_R, d), lambda b: (b, 0)),
            pl.BlockSpec((_R // 32, d), lambda b: (b, 0)),
            pl.BlockSpec((1, d), lambda b: (0, 0)),
            pl.BlockSpec((nvp, d), lambda b: (0, 0)),
            pl.BlockSpec(memory_space=pltpu.SMEM),
            pl.BlockSpec(memory_space=pltpu.SMEM),
        ],
        out_specs=pl.BlockSpec((_R, d), lambda b: (b, 0)),
        out_shape=jax.ShapeDtypeStruct((num_nodes, d), x.dtype),
    )(x, maskp, tok, noise_vals, lo, dst_local)


def kernel(adj, x, enc_mask_token):
    p = _plan(x.shape[0])
    noise_vals = _sc_gather_rows(x, jnp.asarray(p["nsrc"]), p["nvp"])
    out_x = _tc_select_patch(
        x, jnp.asarray(p["maskp"]), enc_mask_token, noise_vals,
        jnp.asarray(p["lo"]), jnp.asarray(p["dst_local"]),
    )
    return (adj, out_x, jnp.asarray(p["mask_nodes"]), jnp.asarray(p["keep_nodes"]))
